# baseline (speedup 1.0000x reference)
import functools

import jax
import jax.numpy as jnp
from jax import lax
from jax.experimental import pallas as pl
from jax.experimental.pallas import tpu as pltpu
from jax.experimental.pallas import tpu_sc as plsc

D_MODEL = 64
T2V_IN = 4
T2V_DIM = 16
HIDDEN = 32
TN = 4096


def _fast_sin(x):
    f32 = jnp.float32
    n = jnp.floor(x * f32(0.15915494309189535) + f32(0.5))
    r = x - n * f32(6.283185307179586)
    r2 = r * r
    p = f32(-1.51639979e-04)
    p = p * r2 + f32(8.07014720e-03)
    p = p * r2 + f32(-1.66209727e-01)
    p = p * r2 + f32(9.99919463e-01)
    return r * p


def _sc_id_gather(table, idx):
    info = plsc.get_sparse_core_info()
    NC, NS = info.num_cores, info.num_subcores
    NW = NC * NS
    N = idx.shape[0]
    b_w = N // NW
    C = 512
    n_ch = b_w // C
    mesh = plsc.VectorSubcoreMesh(core_axis_name="c", subcore_axis_name="s")

    @functools.partial(
        pl.kernel, mesh=mesh,
        out_type=jax.ShapeDtypeStruct((N, 128), jnp.float32),
        scratch_types=[
            pltpu.VMEM((C,), jnp.int32),
            pltpu.VMEM((C, 128), jnp.float32),
            pltpu.SemaphoreType.DMA,
        ],
    )
    def k(table_hbm, idx_hbm, out_hbm, idx_v, rows_v, sem):
        wid = lax.axis_index("s") * NC + lax.axis_index("c")
        base = wid * b_w

        def chunk(ci, carry):
            off = base + ci * C
            pltpu.sync_copy(idx_hbm.at[pl.ds(off, C)], idx_v)
            pltpu.async_copy(table_hbm.at[idx_v], rows_v, sem).wait()
            pltpu.sync_copy(rows_v, out_hbm.at[pl.ds(off, C)])
            return carry

        lax.fori_loop(0, n_ch, chunk, 0)

    return k(table, idx)


def _body(yb, xb, idp_ref, poswb_ref, s3w, evnt_pad,
          m1, b1c, w2c, b2s, out_ref):
    f32 = jnp.float32
    y = yb[...]
    x3 = xb[...]
    tok = y.shape[0]

    aff = jnp.dot(x3, s3w[...], preferred_element_type=f32) + poswb_ref[...]
    ch = jax.lax.broadcasted_iota(jnp.int32, (tok, D_MODEL), 1)
    t2v = jnp.where(ch % T2V_DIM == 0, aff, _fast_sin(aff))

    bf16 = jnp.bfloat16
    yidx = jnp.floor(y[:, 4:7]).astype(bf16)
    lane128 = jax.lax.broadcasted_iota(jnp.int32, (tok, 128), 1).astype(bf16)
    one = jnp.ones((), bf16)
    zero = jnp.zeros((), bf16)
    cnt_evt = (jnp.where(yidx[:, 0:1] == lane128, one, zero)
               + jnp.where(yidx[:, 2:3] == lane128, one, zero))
    evt_emb = jnp.dot(cnt_evt, evnt_pad[...], preferred_element_type=f32)
    id_emb = idp_ref[:, 0:D_MODEL]

    h = jax.nn.relu(jnp.dot(y[:, 0:5], m1[...], preferred_element_type=f32)
                    + b1c[...])
    tv = jnp.dot(h, w2c[...], preferred_element_type=f32) + b2s[...]

    out_ref[...] = t2v + evt_emb + id_emb + tv


def kernel(y, x, t2v_w, t2v_b, evnt_table, id_table,
           ffn0_w1, ffn0_b1, ffn0_w2, ffn0_b2,
           ffn1_w1, ffn1_b1, ffn1_w2, ffn1_b2,
           ffn2_w1, ffn2_b1, ffn2_w2, ffn2_b2,
           ffn3_w1, ffn3_b1, ffn3_w2, ffn3_b2):
    bs, L, _ = y.shape
    f32 = jnp.float32

    wflat = t2v_w.reshape(D_MODEL)
    bflat = t2v_b.reshape(D_MODEL)
    ch = jnp.arange(D_MODEL)
    grp = ch // T2V_DIM
    s3w = jnp.where(grp[None, :] == jnp.arange(3)[:, None], wflat[None, :], 0.0)
    pos = (jnp.arange(L, dtype=f32) / L)[:, None]
    poswb = jnp.where(grp[None, :] == 3, pos * wflat[None, :], 0.0) + bflat[None, :]

    evnt_pad = jnp.pad(evnt_table, ((0, 128 - evnt_table.shape[0]), (0, 0))).astype(jnp.bfloat16)

    w1s = [ffn0_w1, ffn1_w1, ffn2_w1, ffn3_w1]
    w1a = jnp.concatenate([w[0] for w in w1s])
    w1b = jnp.concatenate([w[1] for w in w1s])
    hgrp = jnp.arange(4 * HIDDEN) // HIDDEN
    m1 = jnp.concatenate(
        [jnp.where(hgrp[None, :] == jnp.arange(4)[:, None], w1b[None, :], 0.0),
         w1a[None, :]], axis=0)
    b1c = jnp.concatenate([ffn0_b1, ffn1_b1, ffn2_b1, ffn3_b1]).reshape(1, 4 * HIDDEN)
    w2c = jnp.concatenate([ffn0_w2, ffn1_w2, ffn2_w2, ffn3_w2], axis=0)
    b2s = (ffn0_b2 + ffn1_b2 + ffn2_b2 + ffn3_b2).reshape(1, D_MODEL)

    N = bs * L
    nlb = L // TN
    y2 = y.reshape(N, 7)
    x2 = x.reshape(N, 3)

    idc = jnp.floor(y2[:, 5]).astype(jnp.int32)
    id_pad128 = jnp.pad(id_table, ((0, 0), (0, 128 - D_MODEL)))
    idp = _sc_id_gather(id_pad128, idc)

    full = lambda shape: pl.BlockSpec(shape, lambda g: (0, 0))
    emb = pl.pallas_call(
        _body,
        grid=(N // TN,),
        in_specs=[
            pl.BlockSpec((TN, 7), lambda g: (g, 0)),
            pl.BlockSpec((TN, 3), lambda g: (g, 0)),
            pl.BlockSpec((TN, 128), lambda g: (g, 0)),
            pl.BlockSpec((TN, D_MODEL), lambda g: (g % nlb, 0)),
            full((3, D_MODEL)),
            full((128, D_MODEL)),
            full((5, 4 * HIDDEN)), full((1, 4 * HIDDEN)),
            full((128, D_MODEL)), full((1, D_MODEL)),
        ],
        out_specs=pl.BlockSpec((TN, D_MODEL), lambda g: (g, 0)),
        out_shape=jax.ShapeDtypeStruct((N, D_MODEL), f32),
    )(y2, x2, idp, poswb, s3w, evnt_pad, m1, b1c, w2c, b2s)

    emb = emb.reshape(bs, L, D_MODEL)
    return (emb, jnp.zeros_like(emb))


# speedup vs baseline: 31.3351x; 31.3351x over previous
import functools

import jax
import jax.numpy as jnp
from jax.experimental import pallas as pl

D_MODEL = 64
T2V_IN = 4
T2V_DIM = 16
HIDDEN = 32
TN = 4096


def _fast_sin(x):
    f32 = jnp.float32
    n = jnp.floor(x * f32(0.15915494309189535) + f32(0.5))
    r = x - n * f32(6.283185307179586)
    r2 = r * r
    p = f32(-1.51639979e-04)
    p = p * r2 + f32(8.07014720e-03)
    p = p * r2 + f32(-1.66209727e-01)
    p = p * r2 + f32(9.99919463e-01)
    return r * p


def _body(yb, xb, poswb_ref, s3w, evnt_pad, id_pad,
          m1, b1c, w2c, b2s, out_ref):
    f32 = jnp.float32
    y = yb[...]
    x3 = xb[...]
    tok = y.shape[0]

    aff = jnp.dot(x3, s3w[...], preferred_element_type=f32) + poswb_ref[...]
    ch = jax.lax.broadcasted_iota(jnp.int32, (tok, D_MODEL), 1)
    t2v = jnp.where(ch % T2V_DIM == 0, aff, _fast_sin(aff))

    bf16 = jnp.bfloat16
    yidx = jnp.floor(y[:, 4:7]).astype(bf16)
    lane128 = jax.lax.broadcasted_iota(jnp.int32, (tok, 128), 1).astype(bf16)
    one = jnp.ones((), bf16)
    zero = jnp.zeros((), bf16)
    cnt_evt = (jnp.where(yidx[:, 0:1] == lane128, one, zero)
               + jnp.where(yidx[:, 2:3] == lane128, one, zero))
    oh_id = jnp.where(yidx[:, 1:2] == lane128, one, zero)
    evt_emb = jnp.dot(cnt_evt, evnt_pad[...], preferred_element_type=f32)
    id_emb = jnp.dot(oh_id, id_pad[...], preferred_element_type=f32)

    h = jax.nn.relu(jnp.dot(y[:, 0:5], m1[...], preferred_element_type=f32)
                    + b1c[...])
    tv = jnp.dot(h, w2c[...], preferred_element_type=f32) + b2s[...]

    out_ref[...] = t2v + evt_emb + id_emb + tv


def kernel(y, x, t2v_w, t2v_b, evnt_table, id_table,
           ffn0_w1, ffn0_b1, ffn0_w2, ffn0_b2,
           ffn1_w1, ffn1_b1, ffn1_w2, ffn1_b2,
           ffn2_w1, ffn2_b1, ffn2_w2, ffn2_b2,
           ffn3_w1, ffn3_b1, ffn3_w2, ffn3_b2):
    bs, L, _ = y.shape
    f32 = jnp.float32

    wflat = t2v_w.reshape(D_MODEL)
    bflat = t2v_b.reshape(D_MODEL)
    ch = jnp.arange(D_MODEL)
    grp = ch // T2V_DIM
    s3w = jnp.where(grp[None, :] == jnp.arange(3)[:, None], wflat[None, :], 0.0)
    pos = (jnp.arange(L, dtype=f32) / L)[:, None]
    poswb = jnp.where(grp[None, :] == 3, pos * wflat[None, :], 0.0) + bflat[None, :]

    evnt_pad = jnp.pad(evnt_table, ((0, 128 - evnt_table.shape[0]), (0, 0))).astype(jnp.bfloat16)
    id_pad = jnp.pad(id_table, ((0, 128 - id_table.shape[0]), (0, 0))).astype(jnp.bfloat16)

    w1s = [ffn0_w1, ffn1_w1, ffn2_w1, ffn3_w1]
    w1a = jnp.concatenate([w[0] for w in w1s])
    w1b = jnp.concatenate([w[1] for w in w1s])
    hgrp = jnp.arange(4 * HIDDEN) // HIDDEN
    m1 = jnp.concatenate(
        [jnp.where(hgrp[None, :] == jnp.arange(4)[:, None], w1b[None, :], 0.0),
         w1a[None, :]], axis=0)
    b1c = jnp.concatenate([ffn0_b1, ffn1_b1, ffn2_b1, ffn3_b1]).reshape(1, 4 * HIDDEN)
    w2c = jnp.concatenate([ffn0_w2, ffn1_w2, ffn2_w2, ffn3_w2], axis=0)
    b2s = (ffn0_b2 + ffn1_b2 + ffn2_b2 + ffn3_b2).reshape(1, D_MODEL)

    N = bs * L
    nlb = L // TN
    y2 = y.reshape(N, 7)
    x2 = x.reshape(N, 3)
    full = lambda shape: pl.BlockSpec(shape, lambda g: (0, 0))
    emb = pl.pallas_call(
        _body,
        grid=(N // TN,),
        in_specs=[
            pl.BlockSpec((TN, 7), lambda g: (g, 0)),
            pl.BlockSpec((TN, 3), lambda g: (g, 0)),
            pl.BlockSpec((TN, D_MODEL), lambda g: (g % nlb, 0)),
            full((3, D_MODEL)),
            full((128, D_MODEL)), full((128, D_MODEL)),
            full((5, 4 * HIDDEN)), full((1, 4 * HIDDEN)),
            full((128, D_MODEL)), full((1, D_MODEL)),
        ],
        out_specs=pl.BlockSpec((TN, D_MODEL), lambda g: (g, 0)),
        out_shape=jax.ShapeDtypeStruct((N, D_MODEL), f32),
    )(y2, x2, poswb, s3w, evnt_pad, id_pad, m1, b1c, w2c, b2s)

    emb = emb.reshape(bs, L, D_MODEL)
    return (emb, jnp.zeros_like(emb))
